# initial kernel scaffold (unmeasured)
import jax
import jax.numpy as jnp
from jax import lax
from jax.experimental import pallas as pl
from jax.experimental.pallas import tpu as pltpu

N_DEV = 32
B = 128
D = 128
H = 256


def kernel(x, Win0, Wout0, Win1, Wout1, Win2, Wout2):
    def body(x_ref, win0, wout0, win1, wout1, win2, wout2, out_ref,
             xg, pg, ps, send_sems, recv_sems):
        my = lax.axis_index("i")
        n = jnp.int32(N_DEV)

        def exchange(p, src_buf, dst_buf, src_is_dst_slot):
            def mk(off):
                dst = lax.rem(my + off, n)
                slot = dst if src_is_dst_slot else my
                return pltpu.make_async_remote_copy(
                    src_ref=src_buf.at[slot],
                    dst_ref=dst_buf.at[my],
                    send_sem=send_sems.at[p, off],
                    recv_sem=recv_sems.at[p, my],
                    device_id=(dst,),
                    device_id_type=pl.DeviceIdType.MESH,
                )

            def send_body(off, c):
                mk(off).start()
                return c
            lax.fori_loop(1, N_DEV, send_body, 0)

            def recv_body(off, c):
                src = lax.rem(my + off, n)
                r = pltpu.make_async_remote_copy(
                    src_ref=src_buf.at[my],
                    dst_ref=dst_buf.at[src],
                    send_sem=send_sems.at[p, off],
                    recv_sem=recv_sems.at[p, src],
                    device_id=(my,),
                    device_id_type=pl.DeviceIdType.MESH,
                )
                r.wait_recv()
                return c
            lax.fori_loop(1, N_DEV, recv_body, 0)

            def sendwait_body(off, c):
                mk(off).wait_send()
                return c
            lax.fori_loop(1, N_DEV, sendwait_body, 0)

        def layer(win_ref, wout_ref):
            exchange(0, xg, xg, src_is_dst_slot=False)
            x_full = xg[...].reshape(N_DEV * B, D)
            h = jnp.dot(x_full, win_ref[...].astype(jnp.bfloat16),
                        preferred_element_type=jnp.float32)
            h = jnp.maximum(h, 0.0).astype(jnp.bfloat16)
            partial = jnp.dot(h, wout_ref[...].astype(jnp.bfloat16),
                              preferred_element_type=jnp.float32)
            ps[...] = partial.reshape(N_DEV, B, D).astype(jnp.bfloat16)
            own = lax.dynamic_slice(partial, (my * B, 0), (B, D))
            pg[pl.ds(my, 1)] = own.astype(jnp.bfloat16).reshape(1, B, D)
            exchange(1, ps, pg, src_is_dst_slot=True)
            x_new = jnp.sum(pg[...].astype(jnp.float32), axis=0)
            xg[pl.ds(my, 1)] = x_new.astype(jnp.bfloat16).reshape(1, B, D)
            return x_new

        xg[pl.ds(my, 1)] = x_ref[...].astype(jnp.bfloat16).reshape(1, B, D)
        layer(win0, wout0)
        layer(win1, wout1)
        x_out = layer(win2, wout2)
        out_ref[...] = x_out

    return pl.pallas_call(
        body,
        out_shape=jax.ShapeDtypeStruct((B, D), jnp.float32),
        in_specs=[pl.BlockSpec(memory_space=pltpu.VMEM)] * 7,
        out_specs=pl.BlockSpec(memory_space=pltpu.VMEM),
        scratch_shapes=[
            pltpu.VMEM((N_DEV, B, D), jnp.bfloat16),
            pltpu.VMEM((N_DEV, B, D), jnp.bfloat16),
            pltpu.VMEM((N_DEV, B, D), jnp.bfloat16),
            pltpu.SemaphoreType.DMA((2, N_DEV)),
            pltpu.SemaphoreType.DMA((2, N_DEV)),
        ],
        compiler_params=pltpu.CompilerParams(collective_id=0),
    )(x, Win0, Wout0, Win1, Wout1, Win2, Wout2)


# baseline (device time: 109431 ns/iter reference)
import jax
import jax.numpy as jnp
from jax import lax
from jax.experimental import pallas as pl
from jax.experimental.pallas import tpu as pltpu

N_DEV = 32
B = 128
D = 128
H = 256


def kernel(x, Win0, Wout0, Win1, Wout1, Win2, Wout2):
    def body(x_ref, win0, wout0, win1, wout1, win2, wout2, out_ref,
             xg, pg, ps, send_sems, recv_sems):
        my = lax.axis_index("i")
        n = jnp.int32(N_DEV)

        def exchange(p, src_buf, dst_buf, src_is_dst_slot):
            def mk(off):
                dst = lax.rem(my + off, n)
                slot = dst if src_is_dst_slot else my
                return pltpu.make_async_remote_copy(
                    src_ref=src_buf.at[slot],
                    dst_ref=dst_buf.at[my],
                    send_sem=send_sems.at[p, off],
                    recv_sem=recv_sems.at[p, my],
                    device_id=(dst,),
                    device_id_type=pl.DeviceIdType.MESH,
                )

            def send_body(off, c):
                mk(off).start()
                return c
            lax.fori_loop(1, N_DEV, send_body, 0)

            def recv_body(off, c):
                src = lax.rem(my + off, n)
                r = pltpu.make_async_remote_copy(
                    src_ref=src_buf.at[my],
                    dst_ref=dst_buf.at[src],
                    send_sem=send_sems.at[p, off],
                    recv_sem=recv_sems.at[p, src],
                    device_id=(my,),
                    device_id_type=pl.DeviceIdType.MESH,
                )
                r.wait_recv()
                return c
            lax.fori_loop(1, N_DEV, recv_body, 0)

            def sendwait_body(off, c):
                mk(off).wait_send()
                return c
            lax.fori_loop(1, N_DEV, sendwait_body, 0)

        def layer(win_ref, wout_ref):
            exchange(0, xg, xg, src_is_dst_slot=False)
            x_full = xg[...].reshape(N_DEV * B, D)
            h = jnp.dot(x_full, win_ref[...].astype(jnp.bfloat16),
                        preferred_element_type=jnp.float32)
            h = jnp.maximum(h, 0.0).astype(jnp.bfloat16)
            partial = jnp.dot(h, wout_ref[...].astype(jnp.bfloat16),
                              preferred_element_type=jnp.float32)
            ps[...] = partial.reshape(N_DEV, B, D).astype(jnp.bfloat16)
            pg[pl.ds(my, 1)] = ps[pl.ds(my, 1)]
            exchange(1, ps, pg, src_is_dst_slot=True)
            x_new = jnp.sum(pg[...].astype(jnp.float32), axis=0)
            xg[pl.ds(my, 1)] = x_new.astype(jnp.bfloat16).reshape(1, B, D)
            return x_new

        xg[pl.ds(my, 1)] = x_ref[...].astype(jnp.bfloat16).reshape(1, B, D)
        layer(win0, wout0)
        layer(win1, wout1)
        x_out = layer(win2, wout2)
        out_ref[...] = x_out

    return pl.pallas_call(
        body,
        out_shape=jax.ShapeDtypeStruct((B, D), jnp.float32),
        in_specs=[pl.BlockSpec(memory_space=pltpu.VMEM)] * 7,
        out_specs=pl.BlockSpec(memory_space=pltpu.VMEM),
        scratch_shapes=[
            pltpu.VMEM((N_DEV, B, D), jnp.bfloat16),
            pltpu.VMEM((N_DEV, B, D), jnp.bfloat16),
            pltpu.VMEM((N_DEV, B, D), jnp.bfloat16),
            pltpu.SemaphoreType.DMA((2, N_DEV)),
            pltpu.SemaphoreType.DMA((2, N_DEV)),
        ],
    )(x, Win0, Wout0, Win1, Wout1, Win2, Wout2)


# device time: 105739 ns/iter; 1.0349x vs baseline; 1.0349x over previous
import jax
import jax.numpy as jnp
from jax import lax
from jax.experimental import pallas as pl
from jax.experimental.pallas import tpu as pltpu

N_DEV = 32
B = 128
D = 128
H = 256


def kernel(x, Win0, Wout0, Win1, Wout1, Win2, Wout2):
    def body(x_ref, win0, wout0, win1, wout1, win2, wout2, out_ref,
             xg, pg, ps, send_sems, recv_sems):
        my = lax.axis_index("i")
        n = jnp.int32(N_DEV)

        xg[pl.ds(my, 1)] = x_ref[...].astype(jnp.bfloat16).reshape(1, B, D)

        def layer(win_ref, wout_ref):
            w_in = win_ref[...].astype(jnp.bfloat16)
            w_out = wout_ref[...].astype(jnp.bfloat16)

            def f_chunk(xs):
                h = jnp.dot(xs, w_in, preferred_element_type=jnp.float32)
                h = jnp.maximum(h, 0.0).astype(jnp.bfloat16)
                return jnp.dot(h, w_out, preferred_element_type=jnp.float32)

            def ag_send(off, c):
                dst = lax.rem(my + off, n)
                pltpu.make_async_remote_copy(
                    src_ref=xg.at[my], dst_ref=xg.at[my],
                    send_sem=send_sems.at[0, off], recv_sem=recv_sems.at[0, my],
                    device_id=(dst,), device_id_type=pl.DeviceIdType.MESH,
                ).start()
                return c
            lax.fori_loop(1, N_DEV, ag_send, 0)

            own = f_chunk(xg[pl.ds(my, 1)].reshape(B, D))
            pg[pl.ds(my, 1)] = own.astype(jnp.bfloat16).reshape(1, B, D)

            def chunk_body(off, c):
                src = lax.rem(my + off, n)
                pltpu.make_async_remote_copy(
                    src_ref=xg.at[my], dst_ref=xg.at[src],
                    send_sem=send_sems.at[0, off], recv_sem=recv_sems.at[0, src],
                    device_id=(my,), device_id_type=pl.DeviceIdType.MESH,
                ).wait_recv()
                pc = f_chunk(xg[pl.ds(src, 1)].reshape(B, D))
                ps[pl.ds(src, 1)] = pc.astype(jnp.bfloat16).reshape(1, B, D)
                pltpu.make_async_remote_copy(
                    src_ref=ps.at[src], dst_ref=pg.at[my],
                    send_sem=send_sems.at[1, off], recv_sem=recv_sems.at[1, my],
                    device_id=(src,), device_id_type=pl.DeviceIdType.MESH,
                ).start()
                return c
            lax.fori_loop(1, N_DEV, chunk_body, 0)

            def rs_recv(off, c):
                src = lax.rem(my + off, n)
                pltpu.make_async_remote_copy(
                    src_ref=ps.at[my], dst_ref=pg.at[src],
                    send_sem=send_sems.at[1, off], recv_sem=recv_sems.at[1, src],
                    device_id=(my,), device_id_type=pl.DeviceIdType.MESH,
                ).wait_recv()
                return c
            lax.fori_loop(1, N_DEV, rs_recv, 0)

            x_new = jnp.sum(pg[...].astype(jnp.float32), axis=0)

            def drain(off, c):
                dst = lax.rem(my + off, n)
                pltpu.make_async_remote_copy(
                    src_ref=xg.at[my], dst_ref=xg.at[my],
                    send_sem=send_sems.at[0, off], recv_sem=recv_sems.at[0, my],
                    device_id=(dst,), device_id_type=pl.DeviceIdType.MESH,
                ).wait_send()
                pltpu.make_async_remote_copy(
                    src_ref=ps.at[dst], dst_ref=pg.at[my],
                    send_sem=send_sems.at[1, off], recv_sem=recv_sems.at[1, my],
                    device_id=(dst,), device_id_type=pl.DeviceIdType.MESH,
                ).wait_send()
                return c
            lax.fori_loop(1, N_DEV, drain, 0)

            xg[pl.ds(my, 1)] = x_new.astype(jnp.bfloat16).reshape(1, B, D)
            return x_new

        layer(win0, wout0)
        layer(win1, wout1)
        x_out = layer(win2, wout2)
        out_ref[...] = x_out

    return pl.pallas_call(
        body,
        out_shape=jax.ShapeDtypeStruct((B, D), jnp.float32),
        in_specs=[pl.BlockSpec(memory_space=pltpu.VMEM)] * 7,
        out_specs=pl.BlockSpec(memory_space=pltpu.VMEM),
        scratch_shapes=[
            pltpu.VMEM((N_DEV, B, D), jnp.bfloat16),
            pltpu.VMEM((N_DEV, B, D), jnp.bfloat16),
            pltpu.VMEM((N_DEV, B, D), jnp.bfloat16),
            pltpu.SemaphoreType.DMA((2, N_DEV)),
            pltpu.SemaphoreType.DMA((2, N_DEV)),
        ],
    )(x, Win0, Wout0, Win1, Wout1, Win2, Wout2)


# device time: 104956 ns/iter; 1.0426x vs baseline; 1.0075x over previous
import jax
import jax.numpy as jnp
from jax import lax
from jax.experimental import pallas as pl
from jax.experimental.pallas import tpu as pltpu

N_DEV = 32
B = 128
D = 128
H = 256

GROUPS = [(1, 8), (8, 16), (16, 24), (24, 32)]


def kernel(x, Win0, Wout0, Win1, Wout1, Win2, Wout2):
    def body(x_ref, win0, wout0, win1, wout1, win2, wout2, out_ref,
             xg, pg, ps, send_sems, recv_sems):
        my = lax.axis_index("i")
        n = jnp.int32(N_DEV)

        xg[0:1] = x_ref[...].astype(jnp.bfloat16).reshape(1, B, D)

        def layer(win_ref, wout_ref):
            w_in = win_ref[...].astype(jnp.bfloat16)
            w_out = wout_ref[...].astype(jnp.bfloat16)

            def f_chunks(xs):
                m = xs.shape[0]
                h = jnp.dot(xs.reshape(m * B, D), w_in,
                            preferred_element_type=jnp.float32)
                h = jnp.maximum(h, 0.0).astype(jnp.bfloat16)
                p = jnp.dot(h, w_out, preferred_element_type=jnp.float32)
                return p.reshape(m, B, D)

            for o in range(1, N_DEV):
                pltpu.make_async_remote_copy(
                    src_ref=xg.at[0], dst_ref=xg.at[N_DEV - o],
                    send_sem=send_sems.at[0, o],
                    recv_sem=recv_sems.at[0, N_DEV - o],
                    device_id=(lax.rem(my + o, n),),
                    device_id_type=pl.DeviceIdType.MESH,
                ).start()

            pg[0:1] = f_chunks(xg[0:1]).astype(jnp.bfloat16)

            for lo, hi in GROUPS:
                for o in range(lo, hi):
                    pltpu.make_async_remote_copy(
                        src_ref=xg.at[0], dst_ref=xg.at[o],
                        send_sem=send_sems.at[0, o],
                        recv_sem=recv_sems.at[0, o],
                        device_id=(my,),
                        device_id_type=pl.DeviceIdType.MESH,
                    ).wait_recv()
                ps[lo:hi] = f_chunks(xg[lo:hi]).astype(jnp.bfloat16)
                for o in range(lo, hi):
                    pltpu.make_async_remote_copy(
                        src_ref=ps.at[o], dst_ref=pg.at[N_DEV - o],
                        send_sem=send_sems.at[1, o],
                        recv_sem=recv_sems.at[1, N_DEV - o],
                        device_id=(lax.rem(my + o, n),),
                        device_id_type=pl.DeviceIdType.MESH,
                    ).start()

            for o in range(1, N_DEV):
                pltpu.make_async_remote_copy(
                    src_ref=ps.at[0], dst_ref=pg.at[o],
                    send_sem=send_sems.at[1, o],
                    recv_sem=recv_sems.at[1, o],
                    device_id=(my,),
                    device_id_type=pl.DeviceIdType.MESH,
                ).wait_recv()

            x_new = jnp.sum(pg[...].astype(jnp.float32), axis=0)

            for o in range(1, N_DEV):
                dst = (lax.rem(my + o, n),)
                pltpu.make_async_remote_copy(
                    src_ref=xg.at[0], dst_ref=xg.at[N_DEV - o],
                    send_sem=send_sems.at[0, o],
                    recv_sem=recv_sems.at[0, N_DEV - o],
                    device_id=dst, device_id_type=pl.DeviceIdType.MESH,
                ).wait_send()
                pltpu.make_async_remote_copy(
                    src_ref=ps.at[o], dst_ref=pg.at[N_DEV - o],
                    send_sem=send_sems.at[1, o],
                    recv_sem=recv_sems.at[1, N_DEV - o],
                    device_id=dst, device_id_type=pl.DeviceIdType.MESH,
                ).wait_send()

            xg[0:1] = x_new.astype(jnp.bfloat16).reshape(1, B, D)
            return x_new

        layer(win0, wout0)
        layer(win1, wout1)
        x_out = layer(win2, wout2)
        out_ref[...] = x_out

    return pl.pallas_call(
        body,
        out_shape=jax.ShapeDtypeStruct((B, D), jnp.float32),
        in_specs=[pl.BlockSpec(memory_space=pltpu.VMEM)] * 7,
        out_specs=pl.BlockSpec(memory_space=pltpu.VMEM),
        scratch_shapes=[
            pltpu.VMEM((N_DEV, B, D), jnp.bfloat16),
            pltpu.VMEM((N_DEV, B, D), jnp.bfloat16),
            pltpu.VMEM((N_DEV, B, D), jnp.bfloat16),
            pltpu.SemaphoreType.DMA((2, N_DEV)),
            pltpu.SemaphoreType.DMA((2, N_DEV)),
        ],
    )(x, Win0, Wout0, Win1, Wout1, Win2, Wout2)
